# baseline (speedup 1.0000x reference)
import functools

import jax
import jax.numpy as jnp
from jax import lax
from jax.experimental import pallas as pl
from jax.experimental.pallas import tpu as pltpu
from jax.experimental.pallas import tpu_sc as plsc

ROWS = 128
COLS = 32768
LANES = 16
NUM_WORKERS = 32

SC_ROWS = 96
TC_ROWS = ROWS - SC_ROWS
RPW = SC_ROWS // NUM_WORKERS
VECS_PER_ROW = COLS // LANES

NEG_INF = float("-inf")
NACC = 8


def _insert(a1, a2, a3, v):
    n1 = jnp.maximum(a1, v)
    t = jnp.minimum(a1, v)
    n2 = jnp.maximum(a2, t)
    t2 = jnp.minimum(a2, t)
    n3 = jnp.maximum(a3, t2)
    return n1, n2, n3


def _merge(a, b):
    a1, a2, a3 = a
    b1, b2, b3 = b
    r1 = jnp.maximum(a1, b1)
    c = jnp.minimum(a1, b1)
    d = jnp.maximum(a2, b2)
    r2 = jnp.maximum(c, d)
    r3 = jnp.maximum(jnp.minimum(c, d), jnp.maximum(a3, b3))
    return r1, r2, r3


@functools.partial(
    pl.kernel,
    mesh=plsc.VectorSubcoreMesh(core_axis_name="c", subcore_axis_name="s"),
    out_type=jax.ShapeDtypeStruct((NUM_WORKERS, LANES), jnp.float32),
    compiler_params=pltpu.CompilerParams(
        needs_layout_passes=False, skip_device_barrier=True
    ),
    scratch_types=[
        pltpu.VMEM((COLS,), jnp.float32),
        pltpu.VMEM((COLS,), jnp.float32),
        pltpu.VMEM((LANES,), jnp.float32),
        pltpu.SemaphoreType.DMA,
        pltpu.SemaphoreType.DMA,
        pltpu.SemaphoreType.DMA,
        pltpu.SemaphoreType.DMA,
        pltpu.SemaphoreType.DMA,
        pltpu.SemaphoreType.DMA,
    ],
)
def _topk_sc(x_hbm, out_hbm, buf0, buf1, out_v, sem0, sem1, sc0, sc1, sc2, sc3):
    wid = lax.axis_index("s") * 2 + lax.axis_index("c")
    base_row = wid * RPW
    bufs = (buf0, buf1)
    sems = (sem0, sem1)
    csems = (sc0, sc1, sc2, sc3)
    lanes = lax.iota(jnp.int32, LANES)

    CHUNKS = 4
    CHUNK = COLS // CHUNKS
    row0_copies = [
        pltpu.async_copy(
            x_hbm.at[base_row, pl.ds(c * CHUNK, CHUNK)],
            bufs[0].at[pl.ds(c * CHUNK, CHUNK)],
            csems[c],
        )
        for c in range(CHUNKS)
    ]

    res = jnp.zeros((LANES,), jnp.float32)
    for r in range(RPW):
        if r + 1 < RPW:
            copies_next = pltpu.async_copy(
                x_hbm.at[base_row + (r + 1)],
                bufs[(r + 1) % 2],
                sems[(r + 1) % 2],
            )
        buf = bufs[r % 2]

        def body(i, carry, buf=buf, base=0):
            out = []
            for j in range(NACC):
                a1, a2, a3 = carry[j]
                off = pl.multiple_of(base + (i * NACC + j) * LANES, LANES)
                v = buf[pl.ds(off, LANES)]
                out.append(_insert(a1, a2, a3, v))
            return tuple(out)

        ninf = jnp.full((LANES,), NEG_INF, jnp.float32)
        init = tuple((ninf, ninf, ninf) for _ in range(NACC))
        if r == 0:
            accs = init
            for c in range(CHUNKS):
                row0_copies[c].wait()
                accs = lax.fori_loop(
                    0,
                    CHUNK // LANES // NACC,
                    functools.partial(body, base=c * CHUNK),
                    accs,
                )
            accs = list(accs)
        else:
            copies_prev.wait()
            accs = list(lax.fori_loop(0, VECS_PER_ROW // NACC, body, init))
        if r + 1 < RPW:
            copies_prev = copies_next
        while len(accs) > 1:
            accs = [_merge(accs[i], accs[i + 1]) for i in range(0, len(accs), 2)]
        a1, a2, a3 = accs[0]

        for k in range(3):
            m = jnp.max(a1)
            res = jnp.where(lanes == (3 * r + k), m, res)
            if k < 2:
                f = plsc.all_reduce_ffs(a1 == m)
                sel = lanes == f
                a1 = jnp.where(sel, a2, a1)
                a2 = jnp.where(sel, a3, a2)
                a3 = jnp.where(sel, NEG_INF, a3)

    out_v[...] = res
    pltpu.sync_copy(out_v, out_hbm.at[wid])


TC_BLOCK = 8


def _tc_body(x_ref, o_ref):
    def step(i, carry):
        out = []
        for j in range(NACC):
            a1, a2, a3 = carry[j]
            v = x_ref[:, pl.ds((i * NACC + j) * 128, 128)]
            out.append(_insert(a1, a2, a3, v))
        return tuple(out)

    ninf = jnp.full((TC_BLOCK, 128), NEG_INF, jnp.float32)
    init = tuple((ninf, ninf, ninf) for _ in range(NACC))
    accs = list(lax.fori_loop(0, COLS // 128 // NACC, step, init))
    while len(accs) > 1:
        accs = [_merge(accs[i], accs[i + 1]) for i in range(0, len(accs), 2)]
    a = accs[0]

    for s in (64, 32, 16, 8, 4, 2, 1):
        rolled = tuple(pltpu.roll(v, s, 1) for v in a)
        a = _merge(a, rolled)
    r1, r2, r3 = a
    lane = lax.broadcasted_iota(jnp.int32, (TC_BLOCK, 128), 1)
    full = jnp.where(lane == 0, r1, jnp.where(lane == 1, r2, r3))
    o_ref[...] = full[:, :8]


def _topk_tc(x):
    return pl.pallas_call(
        _tc_body,
        grid=(TC_ROWS // TC_BLOCK,),
        in_specs=[
            pl.BlockSpec((TC_BLOCK, COLS), lambda i: (i + SC_ROWS // TC_BLOCK, 0)),
        ],
        out_specs=pl.BlockSpec((TC_BLOCK, 8), lambda i: (i, 0)),
        out_shape=jax.ShapeDtypeStruct((TC_ROWS, 8), jnp.float32),
    )(x)


def kernel(x):
    sc_part = _topk_sc(x)[:, : 3 * RPW].reshape(SC_ROWS, 3)
    tc_part = _topk_tc(x)[:, :3]
    return jnp.concatenate([sc_part, tc_part], axis=0)


# speedup vs baseline: 1.0382x; 1.0382x over previous
import functools

import jax
import jax.numpy as jnp
from jax import lax
from jax.experimental import pallas as pl
from jax.experimental.pallas import tpu as pltpu
from jax.experimental.pallas import tpu_sc as plsc

ROWS = 128
COLS = 32768
LANES = 16
NUM_WORKERS = 32

SC_ROWS = 64
TC_ROWS = ROWS - SC_ROWS
RPW = SC_ROWS // NUM_WORKERS
VECS_PER_ROW = COLS // LANES

NEG_INF = float("-inf")
NACC = 8


def _insert(a1, a2, a3, v):
    n1 = jnp.maximum(a1, v)
    t = jnp.minimum(a1, v)
    n2 = jnp.maximum(a2, t)
    t2 = jnp.minimum(a2, t)
    n3 = jnp.maximum(a3, t2)
    return n1, n2, n3


def _merge(a, b):
    a1, a2, a3 = a
    b1, b2, b3 = b
    r1 = jnp.maximum(a1, b1)
    c = jnp.minimum(a1, b1)
    d = jnp.maximum(a2, b2)
    r2 = jnp.maximum(c, d)
    r3 = jnp.maximum(jnp.minimum(c, d), jnp.maximum(a3, b3))
    return r1, r2, r3


@functools.partial(
    pl.kernel,
    mesh=plsc.VectorSubcoreMesh(core_axis_name="c", subcore_axis_name="s"),
    out_type=jax.ShapeDtypeStruct((NUM_WORKERS, LANES), jnp.float32),
    compiler_params=pltpu.CompilerParams(
        needs_layout_passes=False, skip_device_barrier=True
    ),
    scratch_types=[
        pltpu.VMEM((COLS,), jnp.float32),
        pltpu.VMEM((COLS,), jnp.float32),
        pltpu.VMEM((LANES,), jnp.float32),
        pltpu.SemaphoreType.DMA,
        pltpu.SemaphoreType.DMA,
        pltpu.SemaphoreType.DMA,
        pltpu.SemaphoreType.DMA,
        pltpu.SemaphoreType.DMA,
        pltpu.SemaphoreType.DMA,
    ],
)
def _topk_sc(x_hbm, out_hbm, buf0, buf1, out_v, sem0, sem1, sc0, sc1, sc2, sc3):
    wid = lax.axis_index("s") * 2 + lax.axis_index("c")
    base_row = wid * RPW
    bufs = (buf0, buf1)
    sems = (sem0, sem1)
    csems = (sc0, sc1, sc2, sc3)
    lanes = lax.iota(jnp.int32, LANES)

    CHUNKS = 4
    CHUNK = COLS // CHUNKS
    row0_copies = [
        pltpu.async_copy(
            x_hbm.at[base_row, pl.ds(c * CHUNK, CHUNK)],
            bufs[0].at[pl.ds(c * CHUNK, CHUNK)],
            csems[c],
        )
        for c in range(CHUNKS)
    ]

    res = jnp.zeros((LANES,), jnp.float32)
    for r in range(RPW):
        if r + 1 < RPW:
            copies_next = pltpu.async_copy(
                x_hbm.at[base_row + (r + 1)],
                bufs[(r + 1) % 2],
                sems[(r + 1) % 2],
            )
        buf = bufs[r % 2]

        def body(i, carry, buf=buf, base=0):
            out = []
            for j in range(NACC):
                a1, a2, a3 = carry[j]
                off = pl.multiple_of(base + (i * NACC + j) * LANES, LANES)
                v = buf[pl.ds(off, LANES)]
                out.append(_insert(a1, a2, a3, v))
            return tuple(out)

        ninf = jnp.full((LANES,), NEG_INF, jnp.float32)
        init = tuple((ninf, ninf, ninf) for _ in range(NACC))
        if r == 0:
            accs = init
            for c in range(CHUNKS):
                row0_copies[c].wait()
                accs = lax.fori_loop(
                    0,
                    CHUNK // LANES // NACC,
                    functools.partial(body, base=c * CHUNK),
                    accs,
                )
            accs = list(accs)
        else:
            copies_prev.wait()
            accs = list(lax.fori_loop(0, VECS_PER_ROW // NACC, body, init))
        if r + 1 < RPW:
            copies_prev = copies_next
        while len(accs) > 1:
            accs = [_merge(accs[i], accs[i + 1]) for i in range(0, len(accs), 2)]
        a1, a2, a3 = accs[0]

        for k in range(3):
            m = jnp.max(a1)
            res = jnp.where(lanes == (3 * r + k), m, res)
            if k < 2:
                f = plsc.all_reduce_ffs(a1 == m)
                sel = lanes == f
                a1 = jnp.where(sel, a2, a1)
                a2 = jnp.where(sel, a3, a2)
                a3 = jnp.where(sel, NEG_INF, a3)

    out_v[...] = res
    pltpu.sync_copy(out_v, out_hbm.at[wid])


TC_BLOCK = 8


def _tc_body(x_ref, o_ref):
    def step(i, carry):
        out = []
        for j in range(NACC):
            a1, a2, a3 = carry[j]
            v = x_ref[:, pl.ds((i * NACC + j) * 128, 128)]
            out.append(_insert(a1, a2, a3, v))
        return tuple(out)

    ninf = jnp.full((TC_BLOCK, 128), NEG_INF, jnp.float32)
    init = tuple((ninf, ninf, ninf) for _ in range(NACC))
    accs = list(lax.fori_loop(0, COLS // 128 // NACC, step, init))
    while len(accs) > 1:
        accs = [_merge(accs[i], accs[i + 1]) for i in range(0, len(accs), 2)]
    a = accs[0]

    for s in (64, 32, 16, 8, 4, 2, 1):
        rolled = tuple(pltpu.roll(v, s, 1) for v in a)
        a = _merge(a, rolled)
    r1, r2, r3 = a
    lane = lax.broadcasted_iota(jnp.int32, (TC_BLOCK, 128), 1)
    full = jnp.where(lane == 0, r1, jnp.where(lane == 1, r2, r3))
    o_ref[...] = full[:, :8]


def _topk_tc(x):
    return pl.pallas_call(
        _tc_body,
        grid=(TC_ROWS // TC_BLOCK,),
        in_specs=[
            pl.BlockSpec((TC_BLOCK, COLS), lambda i: (i + SC_ROWS // TC_BLOCK, 0)),
        ],
        out_specs=pl.BlockSpec((TC_BLOCK, 8), lambda i: (i, 0)),
        out_shape=jax.ShapeDtypeStruct((TC_ROWS, 8), jnp.float32),
    )(x)


def kernel(x):
    sc_part = _topk_sc(x)[:, : 3 * RPW].reshape(SC_ROWS, 3)
    tc_part = _topk_tc(x)[:, :3]
    return jnp.concatenate([sc_part, tc_part], axis=0)
